# baseline (speedup 1.0000x reference)
import functools

import jax
import jax.numpy as jnp
from jax import lax
from jax.experimental import pallas as pl
from jax.experimental.pallas import tpu as pltpu
from jax.experimental.pallas import tpu_sc as plsc

B, N, F_IN, K_NBR, HID = 8, 2048, 64, 64, 128
S = 512
R2 = 0.2 * 0.2
TBL_W = 80
INF = float("inf")


def _pre_body(t_ref, w_ref, b_ref, u_ref):
    u_ref[...] = (
        jnp.dot(t_ref[...], w_ref[...], preferred_element_type=jnp.float32)
        + b_ref[...]
    )


def _run_pre(table, w1pad, b1):
    return pl.pallas_call(
        _pre_body,
        grid=(B,),
        in_specs=[
            pl.BlockSpec((N, TBL_W), lambda i: (i, 0)),
            pl.BlockSpec((TBL_W, HID), lambda i: (0, 0)),
            pl.BlockSpec((1, HID), lambda i: (0, 0)),
        ],
        out_specs=pl.BlockSpec((N, HID), lambda i: (i, 0)),
        out_shape=jax.ShapeDtypeStruct((B * N, HID), jnp.float32),
    )(table, w1pad, b1)


def _fps_body(posT_ref, centers_ref):
    px = posT_ref[0]
    py = posT_ref[1]
    pz = posT_ref[2]
    c0x = px[:, 0]
    c0y = py[:, 0]
    c0z = pz[:, 0]
    centers_ref[0:1] = jnp.stack([c0x, c0y, c0z], axis=0)[None]
    mind0 = (
        (px - c0x[:, None]) ** 2
        + (py - c0y[:, None]) ** 2
        + (pz - c0z[:, None]) ** 2
    )

    lane = lax.broadcasted_iota(jnp.int32, (B, N), 1)

    def body(i, mind):
        nxt = jnp.argmax(mind, axis=1).astype(jnp.int32)
        oh = lane == nxt[:, None]
        cx = jnp.sum(jnp.where(oh, px, 0.0), axis=1)
        cy = jnp.sum(jnp.where(oh, py, 0.0), axis=1)
        cz = jnp.sum(jnp.where(oh, pz, 0.0), axis=1)
        centers_ref[pl.ds(i, 1)] = jnp.stack([cx, cy, cz], axis=0)[None]
        d = (px - cx[:, None]) ** 2 + (py - cy[:, None]) ** 2 + (pz - cz[:, None]) ** 2
        return jnp.minimum(mind, d)

    lax.fori_loop(1, S, body, mind0)


def _run_fps(posT):
    return pl.pallas_call(
        _fps_body,
        out_shape=jax.ShapeDtypeStruct((S, 3, B), jnp.float32),
    )(posT)


BS2 = 128
R2BITS = 1025758986
CH = 128
NCH = N // CH


def _cumsum_lanes(v):
    r = lax.broadcasted_iota(jnp.int32, (CH, CH), 0)
    c = lax.broadcasted_iota(jnp.int32, (CH, CH), 1)
    tri = (r <= c).astype(jnp.float32)
    cs = jnp.dot(
        v.reshape(BS2 * NCH, CH), tri, preferred_element_type=jnp.float32
    ).reshape(BS2, NCH, CH)
    tot = cs[:, :, CH - 1]
    r16 = lax.broadcasted_iota(jnp.int32, (NCH, NCH), 0)
    c16 = lax.broadcasted_iota(jnp.int32, (NCH, NCH), 1)
    tri16 = (r16 < c16).astype(jnp.float32)
    off = jnp.dot(tot, tri16, preferred_element_type=jnp.float32)
    return cs + off[:, :, None]


def _ballq_body(boff, posT_ref, centers_ref, nbr_ref):
    b = pl.program_id(0) + boff
    pb = posT_ref[0]
    cb = centers_ref[0]
    d2 = (
        (cb[0][:, None, None] - pb[0][None]) ** 2
        + (cb[1][:, None, None] - pb[1][None]) ** 2
        + (cb[2][:, None, None] - pb[2][None]) ** 2
    )
    dd = jnp.where(d2 <= R2, d2, INF)
    ibits = lax.bitcast_convert_type(dd, jnp.int32)
    base = b * N
    lane = lax.broadcasted_iota(jnp.int32, (BS2, NCH, CH), 1) * CH + (
        lax.broadcasted_iota(jnp.int32, (BS2, NCH, CH), 2)
    )
    lanef = lane.astype(jnp.float32)

    def sbody(i, carry):
        lo, hi = carry
        mid = (lo + hi) // 2
        cnt = jnp.sum(
            (ibits <= mid[:, None, None]).astype(jnp.float32), axis=(1, 2)
        )
        ge = cnt >= K_NBR
        return jnp.where(ge, lo, mid + 1), jnp.where(ge, mid, hi)

    _, tbits = lax.fori_loop(
        0,
        31,
        sbody,
        (jnp.zeros((BS2,), jnp.int32), jnp.full((BS2,), R2BITS, jnp.int32)),
    )

    tb = tbits[:, None, None]
    lt = ibits < tb
    ties = ibits == tb
    c_lt = jnp.sum(lt.astype(jnp.float32), axis=(1, 2))
    tie_rank = _cumsum_lanes(ties.astype(jnp.float32))
    sel = lt | (ties & (tie_rank <= (K_NBR - c_lt)[:, None, None]))
    self_f = sel.astype(jnp.float32)
    p = _cumsum_lanes(self_f)
    pp = jnp.where(sel, p, 0.0)
    cnt_sel = jnp.sum(self_f, axis=(1, 2))

    val0 = jnp.sum(jnp.where(pp == 1.0, lanef, 0.0), axis=(1, 2))
    nbr_ref[:, 0:1] = (val0.astype(jnp.int32) + base)[None, None, None, :]

    def body(k, v0):
        kf = k.astype(jnp.float32) * 2.0
        w = jnp.where(pp == kf + 1.0, lanef, 0.0) + jnp.where(
            pp == kf + 2.0, lanef * 4096.0, 0.0
        )
        v = jnp.sum(w, axis=(1, 2))
        hivals = jnp.floor(v * (1.0 / 4096.0))
        lovals = v - hivals * 4096.0
        col0 = jnp.where(kf < cnt_sel, lovals, v0)
        col1 = jnp.where(kf + 1.0 < cnt_sel, hivals, v0)
        nbr_ref[:, pl.ds(2 * k, 1)] = (col0.astype(jnp.int32) + base)[
            None, None, None, :
        ]
        nbr_ref[:, pl.ds(2 * k + 1, 1)] = (col1.astype(jnp.int32) + base)[
            None, None, None, :
        ]
        return v0

    lax.fori_loop(0, K_NBR // 2, body, val0)


def _run_ballq(posT4, centers3, boff):
    nblk = S // BS2
    bh = posT4.shape[0]
    return pl.pallas_call(
        functools.partial(_ballq_body, boff),
        grid=(bh, nblk),
        in_specs=[
            pl.BlockSpec((1, 3, NCH, CH), lambda b, s: (b, 0, 0, 0)),
            pl.BlockSpec((1, 3, BS2), lambda b, s: (b, 0, s)),
        ],
        out_specs=pl.BlockSpec(
            (1, K_NBR, 1, BS2), lambda b, s: (b * nblk + s, 0, 0, 0)
        ),
        out_shape=jax.ShapeDtypeStruct((bh * nblk, K_NBR, 1, BS2), jnp.int32),
    )(posT4, centers3)


_NC, _NS = 2, 16
NW = _NC * _NS
CHUNK = 256


def _sc_gather_body(
    rows_per_w, table_hbm, idx_hbm, out_hbm, idx_v, rows_a, rows_b, sem_g, wb_a, wb_b
):
    wid = lax.axis_index("s") * _NC + lax.axis_index("c")
    base = wid * rows_per_w
    pltpu.sync_copy(idx_hbm.at[pl.ds(base, rows_per_w)], idx_v)

    def pair(i, _):
        c0 = 2 * i * CHUNK
        c1 = c0 + CHUNK

        @pl.when(i > 0)
        def _():
            pltpu.make_async_copy(rows_a, out_hbm.at[pl.ds(base, CHUNK)], wb_a).wait()

        pltpu.async_copy(
            table_hbm.at[idx_v.at[pl.ds(c0, CHUNK)]], rows_a, sem_g
        ).wait()
        pltpu.async_copy(rows_a, out_hbm.at[pl.ds(base + c0, CHUNK)], wb_a)

        @pl.when(i > 0)
        def _():
            pltpu.make_async_copy(rows_b, out_hbm.at[pl.ds(base, CHUNK)], wb_b).wait()

        pltpu.async_copy(
            table_hbm.at[idx_v.at[pl.ds(c1, CHUNK)]], rows_b, sem_g
        ).wait()
        pltpu.async_copy(rows_b, out_hbm.at[pl.ds(base + c1, CHUNK)], wb_b)
        return 0

    lax.fori_loop(0, rows_per_w // (2 * CHUNK), pair, 0)
    pltpu.make_async_copy(rows_a, out_hbm.at[pl.ds(base, CHUNK)], wb_a).wait()
    pltpu.make_async_copy(rows_b, out_hbm.at[pl.ds(base, CHUNK)], wb_b).wait()


def _run_sc_gather(table, idx_flat):
    rows_per_w = idx_flat.shape[0] // NW
    mesh = plsc.VectorSubcoreMesh(core_axis_name="c", subcore_axis_name="s")
    f = functools.partial(
        pl.kernel,
        mesh=mesh,
        out_type=jax.ShapeDtypeStruct((idx_flat.shape[0], HID), jnp.float32),
        scratch_types=[
            pltpu.VMEM((rows_per_w,), jnp.int32),
            pltpu.VMEM((CHUNK, HID), jnp.float32),
            pltpu.VMEM((CHUNK, HID), jnp.float32),
            pltpu.SemaphoreType.DMA,
            pltpu.SemaphoreType.DMA,
            pltpu.SemaphoreType.DMA,
        ],
    )(functools.partial(_sc_gather_body, rows_per_w))
    return f(table, idx_flat)


BSC = 128
ROWS4 = BSC * K_NBR


def _mlp_body(g_ref, wp_ref, w2_ref, b2_ref, cT_ref, out_ref):
    cb = cT_ref[...]
    v = lax.dot_general(
        cb, wp_ref[...], (((0,), (0,)), ((), ())),
        preferred_element_type=jnp.float32,
    )
    h1 = g_ref[...].reshape(BSC, K_NBR, HID) - v[:, None, :]
    h1 = jnp.maximum(h1, 0.0).reshape(ROWS4, HID)
    h2 = jnp.dot(h1, w2_ref[...], preferred_element_type=jnp.float32)
    h2 = jnp.maximum(h2 + b2_ref[...], 0.0)
    out_ref[...] = jnp.max(h2.reshape(BSC, K_NBR, HID), axis=1)


def _run_mlp(g, wpos, w2, b2, centersT):
    nprog = centersT.shape[1] // BSC
    return pl.pallas_call(
        _mlp_body,
        grid=(nprog,),
        in_specs=[
            pl.BlockSpec((ROWS4, HID), lambda i: (i, 0)),
            pl.BlockSpec((3, HID), lambda i: (0, 0)),
            pl.BlockSpec((HID, HID), lambda i: (0, 0)),
            pl.BlockSpec((1, HID), lambda i: (0, 0)),
            pl.BlockSpec((3, BSC), lambda i: (0, i)),
        ],
        out_specs=pl.BlockSpec((BSC, HID), lambda i: (i, 0)),
        out_shape=jax.ShapeDtypeStruct((centersT.shape[1], HID), jnp.float32),
    )(g, wpos, w2, b2, centersT)


NHALF = 4
BH = B // NHALF


def kernel(x, pos, batch, W1, b1, W2, b2):
    pb = pos.reshape(B, N, 3)
    posT = pb.transpose(2, 0, 1)

    centers_sb = _run_fps(posT)

    table = jnp.concatenate(
        [x, pos, jnp.zeros((B * N, TBL_W - F_IN - 3), jnp.float32)], axis=1
    )
    w1pad = jnp.concatenate([W1, jnp.zeros((TBL_W - F_IN - 3, HID), W1.dtype)], 0)
    u = _run_pre(table, w1pad, b1.reshape(1, HID))

    wpos = W1[F_IN : F_IN + 3]
    posT4 = posT.transpose(1, 0, 2).reshape(B, 3, NCH, CH)
    centersT_b = centers_sb.transpose(2, 1, 0)
    xs = []
    for h in range(NHALF):
        lo = h * BH
        nbr = _run_ballq(
            posT4[lo : lo + BH], centersT_b[lo : lo + BH], lo
        )
        idx_flat = (
            nbr.reshape(BH * (S // BS2), K_NBR, BS2).transpose(0, 2, 1).reshape(-1)
        )
        g = _run_sc_gather(u, idx_flat)
        centersT = centers_sb[:, :, lo : lo + BH].transpose(1, 2, 0).reshape(3, BH * S)
        xs.append(_run_mlp(g, wpos, W2, b2.reshape(1, HID), centersT))

    x_new = jnp.concatenate(xs, axis=0)
    pos_new = centers_sb.transpose(2, 0, 1).reshape(B * S, 3)
    batch_new = jnp.repeat(jnp.arange(B, dtype=batch.dtype), S)
    return x_new, pos_new, batch_new


# speedup vs baseline: 1.8155x; 1.8155x over previous
import functools

import jax
import jax.numpy as jnp
from jax import lax
from jax.experimental import pallas as pl
from jax.experimental.pallas import tpu as pltpu
from jax.experimental.pallas import tpu_sc as plsc

B, N, F_IN, K_NBR, HID = 8, 2048, 64, 64, 128
S = 512
R2 = 0.2 * 0.2
TBL_W = 80
INF = float("inf")


def _pre_body(t_ref, w_ref, b_ref, u_ref):
    u_ref[...] = (
        jnp.dot(t_ref[...], w_ref[...], preferred_element_type=jnp.float32)
        + b_ref[...]
    )


def _run_pre(table, w1pad, b1):
    return pl.pallas_call(
        _pre_body,
        grid=(B,),
        in_specs=[
            pl.BlockSpec((N, TBL_W), lambda i: (i, 0)),
            pl.BlockSpec((TBL_W, HID), lambda i: (0, 0)),
            pl.BlockSpec((1, HID), lambda i: (0, 0)),
        ],
        out_specs=pl.BlockSpec((N, HID), lambda i: (i, 0)),
        out_shape=jax.ShapeDtypeStruct((B * N, HID), jnp.float32),
    )(table, w1pad, b1)


def _fps_body(posT_ref, centers_ref):
    px = posT_ref[0]
    py = posT_ref[1]
    pz = posT_ref[2]
    c0x = px[:, 0]
    c0y = py[:, 0]
    c0z = pz[:, 0]
    centers_ref[0:1] = jnp.stack([c0x, c0y, c0z], axis=0)[None]
    mind0 = (
        (px - c0x[:, None]) ** 2
        + (py - c0y[:, None]) ** 2
        + (pz - c0z[:, None]) ** 2
    )

    lane = lax.broadcasted_iota(jnp.int32, (B, N), 1)

    def body(i, mind):
        nxt = jnp.argmax(mind, axis=1).astype(jnp.int32)
        oh = lane == nxt[:, None]
        cx = jnp.sum(jnp.where(oh, px, 0.0), axis=1)
        cy = jnp.sum(jnp.where(oh, py, 0.0), axis=1)
        cz = jnp.sum(jnp.where(oh, pz, 0.0), axis=1)
        centers_ref[pl.ds(i, 1)] = jnp.stack([cx, cy, cz], axis=0)[None]
        d = (px - cx[:, None]) ** 2 + (py - cy[:, None]) ** 2 + (pz - cz[:, None]) ** 2
        return jnp.minimum(mind, d)

    lax.fori_loop(1, S, body, mind0)


def _run_fps(posT):
    return pl.pallas_call(
        _fps_body,
        out_shape=jax.ShapeDtypeStruct((S, 3, B), jnp.float32),
    )(posT)


BS2 = 128
R2BITS = 1025758986
CH = 128
NCH = N // CH


def _cumsum_lanes(v):
    r = lax.broadcasted_iota(jnp.int32, (CH, CH), 0)
    c = lax.broadcasted_iota(jnp.int32, (CH, CH), 1)
    tri = (r <= c).astype(jnp.float32)
    cs = jnp.dot(
        v.reshape(BS2 * NCH, CH), tri, preferred_element_type=jnp.float32
    ).reshape(BS2, NCH, CH)
    tot = jnp.sum(v.reshape(BS2, NCH, CH), axis=2)
    r16 = lax.broadcasted_iota(jnp.int32, (NCH, NCH), 0)
    c16 = lax.broadcasted_iota(jnp.int32, (NCH, NCH), 1)
    tri16 = (r16 < c16).astype(jnp.float32)
    off = jnp.dot(tot, tri16, preferred_element_type=jnp.float32)
    return (cs + off[:, :, None]).reshape(BS2, N)


def _ballq_body(boff, posT_ref, centers_ref, nbr_ref):
    b = pl.program_id(0) + boff
    pb = posT_ref[0]
    cb = centers_ref[0]
    d2 = (
        (cb[0][:, None] - pb[0][None, :]) ** 2
        + (cb[1][:, None] - pb[1][None, :]) ** 2
        + (cb[2][:, None] - pb[2][None, :]) ** 2
    )
    dd = jnp.where(d2 <= R2, d2, INF)
    ibits = lax.bitcast_convert_type(dd, jnp.int32)
    base = b * N
    lane = lax.broadcasted_iota(jnp.int32, (BS2, N), 1)
    lanef = lane.astype(jnp.float32)

    def sbody(i, carry):
        lo, hi = carry
        mid = (lo + hi) // 2
        cnt = jnp.sum((ibits <= mid[:, None]).astype(jnp.float32), axis=1)
        ge = cnt >= K_NBR
        return jnp.where(ge, lo, mid + 1), jnp.where(ge, mid, hi)

    _, tbits = lax.fori_loop(
        0,
        31,
        sbody,
        (jnp.zeros((BS2,), jnp.int32), jnp.full((BS2,), R2BITS, jnp.int32)),
    )

    lt = ibits < tbits[:, None]
    ties = ibits == tbits[:, None]
    c_lt = jnp.sum(lt.astype(jnp.float32), axis=1)
    tie_rank = _cumsum_lanes(ties.astype(jnp.float32))
    sel = lt | (ties & (tie_rank <= (K_NBR - c_lt)[:, None]))
    self_f = sel.astype(jnp.float32)
    p = _cumsum_lanes(self_f)
    pp = jnp.where(sel, p, 0.0)
    cnt_sel = jnp.sum(self_f, axis=1)

    val0 = jnp.sum(jnp.where(pp == 1.0, lanef, 0.0), axis=1)
    nbr_ref[:, 0:1] = (val0.astype(jnp.int32) + base)[None, None, None, :]

    def body(k, v0):
        kf = k.astype(jnp.float32) * 2.0
        w = jnp.where(pp == kf + 1.0, lanef, 0.0) + jnp.where(
            pp == kf + 2.0, lanef * 4096.0, 0.0
        )
        v = jnp.sum(w, axis=1)
        hivals = jnp.floor(v * (1.0 / 4096.0))
        lovals = v - hivals * 4096.0
        col0 = jnp.where(kf < cnt_sel, lovals, v0)
        col1 = jnp.where(kf + 1.0 < cnt_sel, hivals, v0)
        nbr_ref[:, pl.ds(2 * k, 1)] = (col0.astype(jnp.int32) + base)[
            None, None, None, :
        ]
        nbr_ref[:, pl.ds(2 * k + 1, 1)] = (col1.astype(jnp.int32) + base)[
            None, None, None, :
        ]
        return v0

    lax.fori_loop(0, K_NBR // 2, body, val0)


def _run_ballq(posT3, centers3, boff):
    nblk = S // BS2
    bh = posT3.shape[0]
    return pl.pallas_call(
        functools.partial(_ballq_body, boff),
        grid=(bh, nblk),
        in_specs=[
            pl.BlockSpec((1, 3, N), lambda b, s: (b, 0, 0)),
            pl.BlockSpec((1, 3, BS2), lambda b, s: (b, 0, s)),
        ],
        out_specs=pl.BlockSpec(
            (1, K_NBR, 1, BS2), lambda b, s: (b * nblk + s, 0, 0, 0)
        ),
        out_shape=jax.ShapeDtypeStruct((bh * nblk, K_NBR, 1, BS2), jnp.int32),
    )(posT3, centers3)


_NC, _NS = 2, 16
NW = _NC * _NS
CHUNK = 256


def _sc_gather_body(
    rows_per_w, table_hbm, idx_hbm, out_hbm, idx_v, rows_a, rows_b, sem_g, wb_a, wb_b
):
    wid = lax.axis_index("s") * _NC + lax.axis_index("c")
    base = wid * rows_per_w
    pltpu.sync_copy(idx_hbm.at[pl.ds(base, rows_per_w)], idx_v)

    def pair(i, _):
        c0 = 2 * i * CHUNK
        c1 = c0 + CHUNK

        @pl.when(i > 0)
        def _():
            pltpu.make_async_copy(rows_a, out_hbm.at[pl.ds(base, CHUNK)], wb_a).wait()

        pltpu.async_copy(
            table_hbm.at[idx_v.at[pl.ds(c0, CHUNK)]], rows_a, sem_g
        ).wait()
        pltpu.async_copy(rows_a, out_hbm.at[pl.ds(base + c0, CHUNK)], wb_a)

        @pl.when(i > 0)
        def _():
            pltpu.make_async_copy(rows_b, out_hbm.at[pl.ds(base, CHUNK)], wb_b).wait()

        pltpu.async_copy(
            table_hbm.at[idx_v.at[pl.ds(c1, CHUNK)]], rows_b, sem_g
        ).wait()
        pltpu.async_copy(rows_b, out_hbm.at[pl.ds(base + c1, CHUNK)], wb_b)
        return 0

    lax.fori_loop(0, rows_per_w // (2 * CHUNK), pair, 0)
    pltpu.make_async_copy(rows_a, out_hbm.at[pl.ds(base, CHUNK)], wb_a).wait()
    pltpu.make_async_copy(rows_b, out_hbm.at[pl.ds(base, CHUNK)], wb_b).wait()


def _run_sc_gather(table, idx_flat):
    rows_per_w = idx_flat.shape[0] // NW
    mesh = plsc.VectorSubcoreMesh(core_axis_name="c", subcore_axis_name="s")
    f = functools.partial(
        pl.kernel,
        mesh=mesh,
        out_type=jax.ShapeDtypeStruct((idx_flat.shape[0], HID), jnp.float32),
        scratch_types=[
            pltpu.VMEM((rows_per_w,), jnp.int32),
            pltpu.VMEM((CHUNK, HID), jnp.float32),
            pltpu.VMEM((CHUNK, HID), jnp.float32),
            pltpu.SemaphoreType.DMA,
            pltpu.SemaphoreType.DMA,
            pltpu.SemaphoreType.DMA,
        ],
    )(functools.partial(_sc_gather_body, rows_per_w))
    return f(table, idx_flat)


BSC = 128
ROWS4 = BSC * K_NBR


def _mlp_body(g_ref, wp_ref, w2_ref, b2_ref, cT_ref, out_ref):
    cb = cT_ref[...]
    v = lax.dot_general(
        cb, wp_ref[...], (((0,), (0,)), ((), ())),
        preferred_element_type=jnp.float32,
    )
    h1 = g_ref[...].reshape(BSC, K_NBR, HID) - v[:, None, :]
    h1 = jnp.maximum(h1, 0.0).reshape(ROWS4, HID)
    h2 = jnp.dot(h1, w2_ref[...], preferred_element_type=jnp.float32)
    h2 = jnp.maximum(h2 + b2_ref[...], 0.0)
    out_ref[...] = jnp.max(h2.reshape(BSC, K_NBR, HID), axis=1)


def _run_mlp(g, wpos, w2, b2, centersT):
    nprog = centersT.shape[1] // BSC
    return pl.pallas_call(
        _mlp_body,
        grid=(nprog,),
        in_specs=[
            pl.BlockSpec((ROWS4, HID), lambda i: (i, 0)),
            pl.BlockSpec((3, HID), lambda i: (0, 0)),
            pl.BlockSpec((HID, HID), lambda i: (0, 0)),
            pl.BlockSpec((1, HID), lambda i: (0, 0)),
            pl.BlockSpec((3, BSC), lambda i: (0, i)),
        ],
        out_specs=pl.BlockSpec((BSC, HID), lambda i: (i, 0)),
        out_shape=jax.ShapeDtypeStruct((centersT.shape[1], HID), jnp.float32),
    )(g, wpos, w2, b2, centersT)


NHALF = 4
BH = B // NHALF


def kernel(x, pos, batch, W1, b1, W2, b2):
    pb = pos.reshape(B, N, 3)
    posT = pb.transpose(2, 0, 1)

    centers_sb = _run_fps(posT)

    table = jnp.concatenate(
        [x, pos, jnp.zeros((B * N, TBL_W - F_IN - 3), jnp.float32)], axis=1
    )
    w1pad = jnp.concatenate([W1, jnp.zeros((TBL_W - F_IN - 3, HID), W1.dtype)], 0)
    u = _run_pre(table, w1pad, b1.reshape(1, HID))

    wpos = W1[F_IN : F_IN + 3]
    posTb = posT.transpose(1, 0, 2)
    centersT_b = centers_sb.transpose(2, 1, 0)
    xs = []
    for h in range(NHALF):
        lo = h * BH
        nbr = _run_ballq(
            posTb[lo : lo + BH], centersT_b[lo : lo + BH], lo
        )
        idx_flat = (
            nbr.reshape(BH * (S // BS2), K_NBR, BS2).transpose(0, 2, 1).reshape(-1)
        )
        g = _run_sc_gather(u, idx_flat)
        centersT = centers_sb[:, :, lo : lo + BH].transpose(1, 2, 0).reshape(3, BH * S)
        xs.append(_run_mlp(g, wpos, W2, b2.reshape(1, HID), centersT))

    x_new = jnp.concatenate(xs, axis=0)
    pos_new = centers_sb.transpose(2, 0, 1).reshape(B * S, 3)
    batch_new = jnp.repeat(jnp.arange(B, dtype=batch.dtype), S)
    return x_new, pos_new, batch_new


# speedup vs baseline: 1.8201x; 1.0026x over previous
import functools

import jax
import jax.numpy as jnp
from jax import lax
from jax.experimental import pallas as pl
from jax.experimental.pallas import tpu as pltpu
from jax.experimental.pallas import tpu_sc as plsc

B, N, F_IN, K_NBR, HID = 8, 2048, 64, 64, 128
S = 512
R2 = 0.2 * 0.2
TBL_W = 80
INF = float("inf")


def _pre_body(t_ref, w_ref, b_ref, u_ref):
    u_ref[...] = (
        jnp.dot(t_ref[...], w_ref[...], preferred_element_type=jnp.float32)
        + b_ref[...]
    )


def _run_pre(table, w1pad, b1):
    return pl.pallas_call(
        _pre_body,
        grid=(B,),
        in_specs=[
            pl.BlockSpec((N, TBL_W), lambda i: (i, 0)),
            pl.BlockSpec((TBL_W, HID), lambda i: (0, 0)),
            pl.BlockSpec((1, HID), lambda i: (0, 0)),
        ],
        out_specs=pl.BlockSpec((N, HID), lambda i: (i, 0)),
        out_shape=jax.ShapeDtypeStruct((B * N, HID), jnp.float32),
    )(table, w1pad, b1)


def _fps_body(posT_ref, centers_ref):
    px = posT_ref[0]
    py = posT_ref[1]
    pz = posT_ref[2]
    c0x = px[:, 0]
    c0y = py[:, 0]
    c0z = pz[:, 0]
    centers_ref[0:1] = jnp.stack([c0x, c0y, c0z], axis=0)[None]
    mind0 = (
        (px - c0x[:, None]) ** 2
        + (py - c0y[:, None]) ** 2
        + (pz - c0z[:, None]) ** 2
    )

    lane = lax.broadcasted_iota(jnp.int32, (B, N), 1)

    def body(i, mind):
        nxt = jnp.argmax(mind, axis=1).astype(jnp.int32)
        oh = lane == nxt[:, None]
        cx = jnp.sum(jnp.where(oh, px, 0.0), axis=1)
        cy = jnp.sum(jnp.where(oh, py, 0.0), axis=1)
        cz = jnp.sum(jnp.where(oh, pz, 0.0), axis=1)
        centers_ref[pl.ds(i, 1)] = jnp.stack([cx, cy, cz], axis=0)[None]
        d = (px - cx[:, None]) ** 2 + (py - cy[:, None]) ** 2 + (pz - cz[:, None]) ** 2
        return jnp.minimum(mind, d)

    lax.fori_loop(1, S, body, mind0)


def _run_fps(posT):
    return pl.pallas_call(
        _fps_body,
        out_shape=jax.ShapeDtypeStruct((S, 3, B), jnp.float32),
    )(posT)


BS2 = 128
R2BITS = 1025758986
CH = 128
NCH = N // CH


def _cumsum_lanes(v):
    r = lax.broadcasted_iota(jnp.int32, (CH, CH), 0)
    c = lax.broadcasted_iota(jnp.int32, (CH, CH), 1)
    tri = (r <= c).astype(jnp.float32)
    cs = jnp.dot(
        v.reshape(BS2 * NCH, CH), tri, preferred_element_type=jnp.float32
    ).reshape(BS2, NCH, CH)
    tot = cs[:, :, CH - 1]
    r16 = lax.broadcasted_iota(jnp.int32, (NCH, NCH), 0)
    c16 = lax.broadcasted_iota(jnp.int32, (NCH, NCH), 1)
    tri16 = (r16 < c16).astype(jnp.float32)
    off = jnp.dot(tot, tri16, preferred_element_type=jnp.float32)
    return (cs + off[:, :, None]).reshape(BS2, N)


def _ballq_body(boff, posT_ref, centers_ref, nbr_ref):
    b = pl.program_id(0) + boff
    pb = posT_ref[0]
    cb = centers_ref[0]
    d2 = (
        (cb[0][:, None] - pb[0][None, :]) ** 2
        + (cb[1][:, None] - pb[1][None, :]) ** 2
        + (cb[2][:, None] - pb[2][None, :]) ** 2
    )
    dd = jnp.where(d2 <= R2, d2, INF)
    ibits = lax.bitcast_convert_type(dd, jnp.int32)
    base = b * N
    lane = lax.broadcasted_iota(jnp.int32, (BS2, N), 1)
    lanef = lane.astype(jnp.float32)

    def sbody(i, carry):
        lo, hi = carry
        mid = (lo + hi) // 2
        cnt = jnp.sum((ibits <= mid[:, None]).astype(jnp.float32), axis=1)
        ge = cnt >= K_NBR
        return jnp.where(ge, lo, mid + 1), jnp.where(ge, mid, hi)

    _, tbits = lax.fori_loop(
        0,
        31,
        sbody,
        (jnp.zeros((BS2,), jnp.int32), jnp.full((BS2,), R2BITS, jnp.int32)),
    )

    lt = ibits < tbits[:, None]
    ties = ibits == tbits[:, None]
    c_lt = jnp.sum(lt.astype(jnp.float32), axis=1)
    tie_rank = _cumsum_lanes(ties.astype(jnp.float32))
    sel = lt | (ties & (tie_rank <= (K_NBR - c_lt)[:, None]))
    self_f = sel.astype(jnp.float32)
    p = _cumsum_lanes(self_f)
    pp = jnp.where(sel, p, 0.0)
    cnt_sel = jnp.sum(self_f, axis=1)

    val0 = jnp.sum(jnp.where(pp == 1.0, lanef, 0.0), axis=1)
    nbr_ref[:, 0:1] = (val0.astype(jnp.int32) + base)[None, None, None, :]

    def body(k, v0):
        kf = k.astype(jnp.float32) * 2.0
        w = jnp.where(pp == kf + 1.0, lanef, 0.0) + jnp.where(
            pp == kf + 2.0, lanef * 4096.0, 0.0
        )
        v = jnp.sum(w, axis=1)
        hivals = jnp.floor(v * (1.0 / 4096.0))
        lovals = v - hivals * 4096.0
        col0 = jnp.where(kf < cnt_sel, lovals, v0)
        col1 = jnp.where(kf + 1.0 < cnt_sel, hivals, v0)
        nbr_ref[:, pl.ds(2 * k, 1)] = (col0.astype(jnp.int32) + base)[
            None, None, None, :
        ]
        nbr_ref[:, pl.ds(2 * k + 1, 1)] = (col1.astype(jnp.int32) + base)[
            None, None, None, :
        ]
        return v0

    lax.fori_loop(0, K_NBR // 2, body, val0)


def _run_ballq(posT3, centers3, boff):
    nblk = S // BS2
    bh = posT3.shape[0]
    return pl.pallas_call(
        functools.partial(_ballq_body, boff),
        grid=(bh, nblk),
        in_specs=[
            pl.BlockSpec((1, 3, N), lambda b, s: (b, 0, 0)),
            pl.BlockSpec((1, 3, BS2), lambda b, s: (b, 0, s)),
        ],
        out_specs=pl.BlockSpec(
            (1, K_NBR, 1, BS2), lambda b, s: (b * nblk + s, 0, 0, 0)
        ),
        out_shape=jax.ShapeDtypeStruct((bh * nblk, K_NBR, 1, BS2), jnp.int32),
    )(posT3, centers3)


_NC, _NS = 2, 16
NW = _NC * _NS
CHUNK = 256


def _sc_gather_body(
    rows_per_w, table_hbm, idx_hbm, out_hbm, idx_v, rows_a, rows_b, sem_g, wb_a, wb_b
):
    wid = lax.axis_index("s") * _NC + lax.axis_index("c")
    base = wid * rows_per_w
    pltpu.sync_copy(idx_hbm.at[pl.ds(base, rows_per_w)], idx_v)

    def pair(i, _):
        c0 = 2 * i * CHUNK
        c1 = c0 + CHUNK

        @pl.when(i > 0)
        def _():
            pltpu.make_async_copy(rows_a, out_hbm.at[pl.ds(base, CHUNK)], wb_a).wait()

        pltpu.async_copy(
            table_hbm.at[idx_v.at[pl.ds(c0, CHUNK)]], rows_a, sem_g
        ).wait()
        pltpu.async_copy(rows_a, out_hbm.at[pl.ds(base + c0, CHUNK)], wb_a)

        @pl.when(i > 0)
        def _():
            pltpu.make_async_copy(rows_b, out_hbm.at[pl.ds(base, CHUNK)], wb_b).wait()

        pltpu.async_copy(
            table_hbm.at[idx_v.at[pl.ds(c1, CHUNK)]], rows_b, sem_g
        ).wait()
        pltpu.async_copy(rows_b, out_hbm.at[pl.ds(base + c1, CHUNK)], wb_b)
        return 0

    lax.fori_loop(0, rows_per_w // (2 * CHUNK), pair, 0)
    pltpu.make_async_copy(rows_a, out_hbm.at[pl.ds(base, CHUNK)], wb_a).wait()
    pltpu.make_async_copy(rows_b, out_hbm.at[pl.ds(base, CHUNK)], wb_b).wait()


def _run_sc_gather(table, idx_flat):
    rows_per_w = idx_flat.shape[0] // NW
    mesh = plsc.VectorSubcoreMesh(core_axis_name="c", subcore_axis_name="s")
    f = functools.partial(
        pl.kernel,
        mesh=mesh,
        out_type=jax.ShapeDtypeStruct((idx_flat.shape[0], HID), jnp.float32),
        scratch_types=[
            pltpu.VMEM((rows_per_w,), jnp.int32),
            pltpu.VMEM((CHUNK, HID), jnp.float32),
            pltpu.VMEM((CHUNK, HID), jnp.float32),
            pltpu.SemaphoreType.DMA,
            pltpu.SemaphoreType.DMA,
            pltpu.SemaphoreType.DMA,
        ],
    )(functools.partial(_sc_gather_body, rows_per_w))
    return f(table, idx_flat)


BSC = 128
ROWS4 = BSC * K_NBR


def _mlp_body(g_ref, wp_ref, w2_ref, b2_ref, cT_ref, out_ref):
    cb = cT_ref[...]
    v = lax.dot_general(
        cb, wp_ref[...], (((0,), (0,)), ((), ())),
        preferred_element_type=jnp.float32,
    )
    h1 = g_ref[...].reshape(BSC, K_NBR, HID) - v[:, None, :]
    h1 = jnp.maximum(h1, 0.0).reshape(ROWS4, HID)
    h2 = jnp.dot(h1, w2_ref[...], preferred_element_type=jnp.float32)
    h2 = jnp.maximum(h2 + b2_ref[...], 0.0)
    out_ref[...] = jnp.max(h2.reshape(BSC, K_NBR, HID), axis=1)


def _run_mlp(g, wpos, w2, b2, centersT):
    nprog = centersT.shape[1] // BSC
    return pl.pallas_call(
        _mlp_body,
        grid=(nprog,),
        in_specs=[
            pl.BlockSpec((ROWS4, HID), lambda i: (i, 0)),
            pl.BlockSpec((3, HID), lambda i: (0, 0)),
            pl.BlockSpec((HID, HID), lambda i: (0, 0)),
            pl.BlockSpec((1, HID), lambda i: (0, 0)),
            pl.BlockSpec((3, BSC), lambda i: (0, i)),
        ],
        out_specs=pl.BlockSpec((BSC, HID), lambda i: (i, 0)),
        out_shape=jax.ShapeDtypeStruct((centersT.shape[1], HID), jnp.float32),
    )(g, wpos, w2, b2, centersT)


NHALF = 4
BH = B // NHALF


def kernel(x, pos, batch, W1, b1, W2, b2):
    pb = pos.reshape(B, N, 3)
    posT = pb.transpose(2, 0, 1)

    centers_sb = _run_fps(posT)

    table = jnp.concatenate(
        [x, pos, jnp.zeros((B * N, TBL_W - F_IN - 3), jnp.float32)], axis=1
    )
    w1pad = jnp.concatenate([W1, jnp.zeros((TBL_W - F_IN - 3, HID), W1.dtype)], 0)
    u = _run_pre(table, w1pad, b1.reshape(1, HID))

    wpos = W1[F_IN : F_IN + 3]
    posTb = posT.transpose(1, 0, 2)
    centersT_b = centers_sb.transpose(2, 1, 0)
    xs = []
    for h in range(NHALF):
        lo = h * BH
        nbr = _run_ballq(
            posTb[lo : lo + BH], centersT_b[lo : lo + BH], lo
        )
        idx_flat = (
            nbr.reshape(BH * (S // BS2), K_NBR, BS2).transpose(0, 2, 1).reshape(-1)
        )
        g = _run_sc_gather(u, idx_flat)
        centersT = centers_sb[:, :, lo : lo + BH].transpose(1, 2, 0).reshape(3, BH * S)
        xs.append(_run_mlp(g, wpos, W2, b2.reshape(1, HID), centersT))

    x_new = jnp.concatenate(xs, axis=0)
    pos_new = centers_sb.transpose(2, 0, 1).reshape(B * S, 3)
    batch_new = jnp.repeat(jnp.arange(B, dtype=batch.dtype), S)
    return x_new, pos_new, batch_new
